# baseline (speedup 1.0000x reference)
import jax
import jax.numpy as jnp
from jax import lax
from jax.experimental import pallas as pl
from jax.experimental.pallas import tpu as pltpu
from jax.experimental.pallas import tpu_sc as plsc

_B, _C, _H, _W = 8, 255, 76, 76
_S = _H * _W
_HS = _S // 2
_COLS = 2 * _W
_NT = _S // _COLS
_ZROW = _COLS * 16
_ZROW15 = _COLS * 15
_SLAB = _NT * _ZROW
_IMG = _S * _C
_PW = (10.0 / 608.0, 16.0 / 608.0, 33.0 / 608.0)
_PH = (13.0 / 608.0, 30.0 / 608.0, 23.0 / 608.0)


def _slab_consts(slab):
    is_exp, scale, inv, d0, d1 = [], [], [], [], []
    for lane in range(16):
        c = min(slab * 16 + lane, _C - 1)
        a, d = c // 85, c % 85
        is_exp.append(d in (2, 3))
        scale.append(_PW[a] if d == 2 else (_PH[a] if d == 3 else 0.0))
        inv.append(1.0 / _W if d in (0, 1) else (0.0 if d in (2, 3) else 1.0))
        d0.append(1.0 if d == 0 else 0.0)
        d1.append(1.0 if d == 1 else 0.0)
    return is_exp, scale, inv, d0, d1


def _lane_vec(vals, iota):
    uniq = sorted(set(vals))
    out = jnp.full((16,), jnp.float32(uniq[0]))
    for u in uniq[1:]:
        mask = jnp.zeros((16,), jnp.bool_)
        for lane, v in enumerate(vals):
            if v == u:
                mask = mask | (iota == lane)
        out = jnp.where(mask, jnp.float32(u), out)
    return out


def _decode_body(x_ref, y_ref, zx_ref, inrow, rbuf, buf2, outb, sem):
    core = lax.axis_index("c")
    sid = lax.axis_index("s")
    iota = lax.iota(jnp.int32, 16)
    viota16 = iota * 16
    viota15 = iota * 15

    def scratch_region(b_local):
        if b_local in (0, 3):
            return zx_ref.at[core, :]
        return y_ref.at[core * 4 + b_local + 1, :]

    for b_local in range(4):
        bg = core * 4 + b_local
        scr = scratch_region(b_local)

        for half in range(2):
            @pl.when(sid != 15)
            def _():
                for r in range(16):
                    pltpu.sync_copy(
                        x_ref.at[bg, sid * 16 + r, pl.ds(half * _HS, _HS)],
                        inrow)

                    @plsc.parallel_loop(0, _HS // 16 + 1, 1, unroll=2)
                    def rk(k):
                        off = jnp.minimum(16 * k, _HS - 16)
                        v = inrow[pl.ds(off, 16)]
                        plsc.store_scatter(rbuf,
                                           [viota16 + (off * 16 + r)], v)

                pltpu.sync_copy(
                    rbuf.at[pl.ds(0, _HS * 16)],
                    scr.at[pl.ds(sid * _SLAB + half * (_HS * 16), _HS * 16)])

            @pl.when(sid == 15)
            def _():
                for r in range(15):
                    pltpu.sync_copy(
                        x_ref.at[bg, 240 + r, pl.ds(half * _HS, _HS)],
                        inrow)

                    @plsc.parallel_loop(0, _HS // 16 + 1, 1, unroll=2)
                    def rk(k):
                        off = jnp.minimum(16 * k, _HS - 16)
                        v = inrow[pl.ds(off, 16)]
                        plsc.store_scatter(rbuf,
                                           [viota15 + (off * 15 + r)], v)

                pltpu.sync_copy(
                    rbuf.at[pl.ds(0, _HS * 15)],
                    scr.at[pl.ds(15 * _SLAB + half * (_HS * 15), _HS * 15)])

        plsc.subcore_barrier()

        def do_tile(j):
            descs = [
                pltpu.async_copy(
                    scr.at[pl.ds(slab * _SLAB + j * _ZROW, _ZROW)],
                    buf2.at[slab, :],
                    sem,
                )
                for slab in range(15)
            ]
            descs.append(pltpu.async_copy(
                scr.at[pl.ds(15 * _SLAB + j * _ZROW15, _ZROW15)],
                buf2.at[15, pl.ds(0, _ZROW15)],
                sem,
            ))
            for dsc in descs:
                dsc.wait()

            def col(s, carry):
                wf = jnp.where(s < _W, s, s - _W).astype(jnp.float32)
                hf = (2 * j + jnp.where(s < _W, 0, 1)).astype(jnp.float32)
                vs = [buf2[slab, pl.ds(s * 16, 16)] for slab in range(15)]
                vs.append(buf2[15, pl.ds(s * 15, 16)])
                res = []
                for slab in range(16):
                    v = vs[slab]
                    is_exp, scale, inv, d0, d1 = _slab_consts(slab)
                    sig = 1.0 / (1.0 + jnp.exp(-v))
                    if not any(is_exp):
                        res.append(sig)
                        continue
                    e = jnp.exp(v)
                    mexp = _lane_vec([1.0 if t else 0.0 for t in is_exp],
                                     iota) > 0.5
                    addv = (_lane_vec(d0, iota) * wf
                            + _lane_vec(d1, iota) * hf)
                    r = jnp.where(mexp, _lane_vec(scale, iota) * e,
                                  (sig + addv) * _lane_vec(inv, iota))
                    res.append(r)
                for slab in range(16):
                    outb[pl.ds(s * _C + slab * 16, 16)] = res[slab]
                return carry

            lax.fori_loop(0, _COLS, col, 0)
            pltpu.sync_copy(
                outb.at[pl.ds(0, _COLS * _C)],
                y_ref.at[bg, pl.ds(j * _COLS * _C, _COLS * _C)],
            )

        trip = jnp.where(sid < 6, 3, 2)

        def jt(jj, carry):
            do_tile(sid + 16 * jj)
            return carry

        lax.fori_loop(0, trip, jt, 0)

        plsc.subcore_barrier()


def kernel(x):
    xr = x.reshape(_B, _C, _S)
    mesh = plsc.VectorSubcoreMesh(core_axis_name="c", subcore_axis_name="s")
    y, _ = pl.kernel(
        _decode_body,
        out_type=(
            jax.ShapeDtypeStruct((_B, _IMG), jnp.float32),
            jax.ShapeDtypeStruct((2, _IMG), jnp.float32),
        ),
        mesh=mesh,
        scratch_types=[
            pltpu.VMEM((_HS,), jnp.float32),
            pltpu.VMEM((_HS * 16,), jnp.float32),
            pltpu.VMEM((16, _ZROW), jnp.float32),
            pltpu.VMEM((_COLS * _C + 16,), jnp.float32),
            pltpu.SemaphoreType.DMA,
        ],
        compiler_params=pltpu.CompilerParams(
            use_tc_tiling_on_sc=False, needs_layout_passes=False),
    )(xr)
    return y.reshape(_B, _IMG // 85, 85)


# speedup vs baseline: 2.0471x; 2.0471x over previous
import jax
import jax.numpy as jnp
from jax import lax
from jax.experimental import pallas as pl
from jax.experimental.pallas import tpu as pltpu
from jax.experimental.pallas import tpu_sc as plsc

_B, _C, _H, _W = 8, 255, 76, 76
_S = _H * _W
_NC, _NS = 2, 16
_COLS = 2 * _W
_NT = _S // _COLS
_PW = (10.0 / 608.0, 16.0 / 608.0, 33.0 / 608.0)
_PH = (13.0 / 608.0, 30.0 / 608.0, 23.0 / 608.0)
_OFFS = (0, 16, 32, 48, 64, 80, 96, 112, 128, 136)


def _decode_body(x_ref, y_ref, inb, idxb, outb, sem):
    wid = lax.axis_index("s") * _NC + lax.axis_index("c")
    b = wid // 4
    q = wid % 4
    start = jnp.where(q < 2, q * 10, 20 + (q - 2) * 9)
    trip = jnp.where(q < 2, 10, 9)
    iota = lax.iota(jnp.int32, 16)

    viota = iota * _C

    def tile(k, carry):
        j = start + k
        s0 = j * _COLS
        row0 = 2 * j
        base = b * (_C * _NT) + j
        for t in range(16):
            coff = jnp.minimum(iota + 16 * t, _C - 1) * _NT
            idxb[pl.ds(16 * t, 16)] = coff + base
        pltpu.async_copy(x_ref.at[idxb], inb, sem).wait()

        for off in _OFFS:
            scv = off + iota
            ge = scv >= _W
            wvf = jnp.where(ge, scv - _W, scv).astype(jnp.float32)
            hvf = (jnp.full((16,), row0, jnp.int32)
                   + ge.astype(jnp.int32)).astype(jnp.float32)
            vx = [inb[85 * a + 0, pl.ds(off, 16)] for a in range(3)]
            vy = [inb[85 * a + 1, pl.ds(off, 16)] for a in range(3)]
            vw = [inb[85 * a + 2, pl.ds(off, 16)] for a in range(3)]
            vh = [inb[85 * a + 3, pl.ds(off, 16)] for a in range(3)]
            sx = [1.0 / (1.0 + jnp.exp(-v)) for v in vx]
            sy = [1.0 / (1.0 + jnp.exp(-v)) for v in vy]
            ew = [jnp.exp(v) for v in vw]
            eh = [jnp.exp(v) for v in vh]
            rx = [(s + wvf) * (1.0 / _W) for s in sx]
            ry = [(s + hvf) * (1.0 / _H) for s in sy]
            rw = [_PW[a] * ew[a] for a in range(3)]
            rh = [_PH[a] * eh[a] for a in range(3)]
            for a in range(3):
                base = off * _C + 85 * a
                plsc.store_scatter(outb, [viota + base], rx[a])
                plsc.store_scatter(outb, [viota + (base + 1)], ry[a])
                plsc.store_scatter(outb, [viota + (base + 2)], rw[a])
                plsc.store_scatter(outb, [viota + (base + 3)], rh[a])

        @plsc.parallel_loop(0, 81, 1, unroll=1)
        def ch(i):
            for a in range(3):
                c = 85 * a + 4 + i
                vs = [inb[c, pl.ds(off, 16)] for off in _OFFS]
                rs = [1.0 / (1.0 + jnp.exp(-v)) for v in vs]
                for off, r in zip(_OFFS, rs):
                    plsc.store_scatter(outb, [viota + (off * _C + c)], r)

        pltpu.sync_copy(outb, y_ref.at[b, pl.ds(s0 * _C, _COLS * _C)])
        return carry

    lax.fori_loop(0, trip, tile, 0)


def kernel(x):
    xr = x.reshape(_B * _C * _NT, _COLS)
    mesh = plsc.VectorSubcoreMesh(core_axis_name="c", subcore_axis_name="s")
    y = pl.kernel(
        _decode_body,
        out_type=jax.ShapeDtypeStruct((_B, _S * _C), jnp.float32),
        mesh=mesh,
        scratch_types=[
            pltpu.VMEM((256, _COLS), jnp.float32),
            pltpu.VMEM((256,), jnp.int32),
            pltpu.VMEM((_COLS * _C,), jnp.float32),
            pltpu.SemaphoreType.DMA,
        ],
        compiler_params=pltpu.CompilerParams(
            use_tc_tiling_on_sc=False, needs_layout_passes=False),
    )(xr)
    return y.reshape(_B, _S * _C // 85, 85)
